# baseline (speedup 1.0000x reference)
import functools

import jax
import jax.numpy as jnp
from jax import lax
from jax.experimental import pallas as pl
from jax.experimental.pallas import tpu as pltpu
from jax.experimental.pallas import tpu_sc as plsc

E = 8
T = 2048
D = 1024
F = 4096
BLK = 128
NB = 40
FT = 1024
NF = F // FT

_info = plsc.get_sparse_core_info()
_NC, _NS = _info.num_cores, _info.num_subcores
_NW = _NC * _NS
_PAIRS_PER_W = 2 * T // _NW
_SCH = 64
_NSCH = _PAIRS_PER_W // _SCH
_TOK_PER_W = T // _NW
_CCH = 16
_NCCH = _TOK_PER_W // _CCH

_sc_mesh = plsc.VectorSubcoreMesh(core_axis_name="c", subcore_axis_name="s")


def _router_body(x_ref, wg_ref, logits_ref, w1_ref, w2_ref, pos1_ref,
                 pos2_ref, be_ref):
    x = x_ref[...]
    Wg = wg_ref[...]
    l = jnp.dot(x, Wg, preferred_element_type=jnp.float32)
    m = jnp.max(l, axis=1, keepdims=True)
    p = jnp.exp(l - m)
    p = p / jnp.sum(p, axis=1, keepdims=True)
    idx8 = lax.broadcasted_iota(jnp.int32, (1, E), 1).astype(jnp.float32)
    v1 = jnp.max(l, axis=1, keepdims=True)
    e1 = jnp.min(jnp.where(l == v1, idx8, float(E)), axis=1, keepdims=True)
    lm = jnp.where(idx8 == e1, -jnp.inf, l)
    v2 = jnp.max(lm, axis=1, keepdims=True)
    e2 = jnp.min(jnp.where(lm == v2, idx8, float(E)), axis=1, keepdims=True)
    oh1 = (idx8 == e1).astype(jnp.float32)
    oh2 = (idx8 == e2).astype(jnp.float32)
    p1 = jnp.sum(oh1 * p, axis=1, keepdims=True)
    p2 = jnp.sum(oh2 * p, axis=1, keepdims=True)
    s = p1 + p2
    w1_ref[...] = p1 / s
    w2_ref[...] = p2 / s
    ii = lax.broadcasted_iota(jnp.int32, (T, T), 0)
    jj = lax.broadcasted_iota(jnp.int32, (T, T), 1)
    Ltri = (jj < ii).astype(jnp.float32)
    oh12 = jnp.concatenate([oh1, oh2], axis=1)
    cum = jnp.dot(Ltri, oh12, preferred_element_type=jnp.float32)
    cum1, cum2 = cum[:, :E], cum[:, E:]
    count1 = jnp.sum(oh1, axis=0, keepdims=True)
    count2 = jnp.sum(oh2, axis=0, keepdims=True)
    count = count1 + count2
    nblk = jnp.floor((count + (BLK - 1)) / BLK)
    ei = lax.broadcasted_iota(jnp.int32, (E, E), 0)
    ej = lax.broadcasted_iota(jnp.int32, (E, E), 1)
    Lo8 = (ei < ej).astype(jnp.float32)
    blk_off = jnp.dot(nblk, Lo8, preferred_element_type=jnp.float32)
    off = BLK * blk_off
    rank1 = jnp.sum(oh1 * cum1, axis=1, keepdims=True)
    rank2 = jnp.sum(oh2 * (count1 + cum2), axis=1, keepdims=True)
    pos1 = jnp.sum(oh1 * off, axis=1, keepdims=True) + rank1
    pos2 = jnp.sum(oh2 * off, axis=1, keepdims=True) + rank2
    bb = lax.broadcasted_iota(jnp.int32, (NB, E), 0).astype(jnp.float32)
    be = jnp.sum((blk_off <= bb).astype(jnp.float32), axis=1, keepdims=True) - 1.0
    be = jnp.clip(be, 0.0, float(E - 1))
    logits_ref[...] = l
    pos1_ref[...] = pos1.astype(jnp.int32)
    pos2_ref[...] = pos2.astype(jnp.int32)
    be_ref[...] = jnp.broadcast_to(be.astype(jnp.int32), (NB, 128))


def _router(x, Wg):
    return pl.pallas_call(
        _router_body,
        out_shape=(
            jax.ShapeDtypeStruct((T, E), jnp.float32),
            jax.ShapeDtypeStruct((T, 1), jnp.float32),
            jax.ShapeDtypeStruct((T, 1), jnp.float32),
            jax.ShapeDtypeStruct((T, 1), jnp.int32),
            jax.ShapeDtypeStruct((T, 1), jnp.int32),
            jax.ShapeDtypeStruct((NB, 128), jnp.int32),
        ),
    )(x, Wg)


@functools.partial(
    pl.kernel,
    mesh=_sc_mesh,
    out_type=(
        jax.ShapeDtypeStruct((NB * BLK, D), jnp.float32),
        jax.ShapeDtypeStruct((NB * BLK,), jnp.float32),
    ),
    scratch_types=[
        pltpu.VMEM((_SCH,), jnp.int32),
        pltpu.VMEM((_SCH,), jnp.float32),
        pltpu.VMEM((_SCH, D), jnp.float32),
        pltpu.SemaphoreType.DMA,
    ],
)
def _sc_scatter(x_hbm, pos_hbm, wflat_hbm, xs_hbm, ws_hbm, idx_v, w_v,
                rows_v, sem):
    wid = lax.axis_index("s") * _NC + lax.axis_index("c")
    for c in range(_NSCH):
        base = wid * _PAIRS_PER_W + c * _SCH
        src = lax.rem(base, T)
        pltpu.sync_copy(pos_hbm.at[pl.ds(base, _SCH)], idx_v)
        pltpu.sync_copy(wflat_hbm.at[pl.ds(base, _SCH)], w_v)
        pltpu.sync_copy(x_hbm.at[pl.ds(src, _SCH)], rows_v)
        pltpu.async_copy(rows_v, xs_hbm.at[idx_v], sem).wait()
        pltpu.async_copy(w_v, ws_hbm.at[idx_v], sem).wait()


def _gmm_body(be_ref, xs_ref, ws_ref, wu_ref, wgt_ref, wd_ref, out_ref):
    f = pl.program_id(1)
    xb = xs_ref[...]
    u = jnp.dot(xb, wu_ref[0], preferred_element_type=jnp.float32)
    g = jnp.dot(xb, wgt_ref[0], preferred_element_type=jnp.float32)
    h = (u * lax.logistic(u)) * g * ws_ref[...]
    part = jnp.dot(h, wd_ref[0], preferred_element_type=jnp.float32)

    @pl.when(f == 0)
    def _():
        out_ref[...] = part

    @pl.when(f != 0)
    def _():
        out_ref[...] += part


def _gmm(be, xs, ws, Wu, Wgate, Wd):
    grid_spec = pltpu.PrefetchScalarGridSpec(
        num_scalar_prefetch=1,
        grid=(NB, NF),
        in_specs=[
            pl.BlockSpec((BLK, D), lambda b, f, be: (b, 0)),
            pl.BlockSpec((BLK, 1), lambda b, f, be: (b, 0)),
            pl.BlockSpec((1, D, FT), lambda b, f, be: (be[b], 0, f)),
            pl.BlockSpec((1, D, FT), lambda b, f, be: (be[b], 0, f)),
            pl.BlockSpec((1, FT, D), lambda b, f, be: (be[b], f, 0)),
        ],
        out_specs=pl.BlockSpec((BLK, D), lambda b, f, be: (b, 0)),
    )
    return pl.pallas_call(
        _gmm_body,
        grid_spec=grid_spec,
        out_shape=jax.ShapeDtypeStruct((NB * BLK, D), jnp.float32),
        compiler_params=pltpu.CompilerParams(
            dimension_semantics=("arbitrary", "arbitrary")),
    )(be, xs, ws, Wu, Wgate, Wd)


@functools.partial(
    pl.kernel,
    mesh=_sc_mesh,
    out_type=jax.ShapeDtypeStruct((T, D), jnp.float32),
    scratch_types=[
        pltpu.VMEM((_CCH,), jnp.int32),
        pltpu.VMEM((_CCH,), jnp.int32),
        pltpu.VMEM((_CCH, D), jnp.float32),
        pltpu.VMEM((_CCH, D), jnp.float32),
        pltpu.VMEM((_CCH, D), jnp.float32),
        pltpu.SemaphoreType.DMA,
    ],
)
def _sc_combine(ys_hbm, pos1_hbm, pos2_hbm, out_hbm,
                idx_a, idx_b, rows_a, rows_b, out_v, sem):
    wid = lax.axis_index("s") * _NC + lax.axis_index("c")
    for c in range(_NCCH):
        base = wid * _TOK_PER_W + c * _CCH
        pltpu.sync_copy(pos1_hbm.at[pl.ds(base, _CCH)], idx_a)
        pltpu.sync_copy(pos2_hbm.at[pl.ds(base, _CCH)], idx_b)
        pltpu.async_copy(ys_hbm.at[idx_a], rows_a, sem).wait()
        pltpu.async_copy(ys_hbm.at[idx_b], rows_b, sem).wait()

        def body(j, carry):
            for v in range(D // 16):
                sl = pl.ds(v * 16, 16)
                out_v[j, sl] = rows_a[j, sl] + rows_b[j, sl]
            return carry

        lax.fori_loop(0, _CCH, body, 0)
        pltpu.sync_copy(out_v, out_hbm.at[pl.ds(base, _CCH)])


def kernel(hidden_states, Wg, Wu, Wgate, Wd):
    x = hidden_states.reshape(-1, D)
    l, w1, w2, pos1, pos2, be2d = _router(x, Wg)
    be = be2d[:, 0]
    pos1f, pos2f = pos1[:, 0], pos2[:, 0]
    pos = jnp.concatenate([pos1f, pos2f])
    wflat = jnp.concatenate([w1[:, 0], w2[:, 0]])
    xs, ws = _sc_scatter(x, pos, wflat)
    ys = _gmm(be, xs, ws.reshape(-1, 1), Wu, Wgate, Wd)
    final = _sc_combine(ys, pos1f, pos2f)
    return final.reshape(1, T, D), l
